# Initial kernel scaffold; baseline (speedup 1.0000x reference)
import jax
import jax.numpy as jnp
from jax.experimental import pallas as pl


def kernel(x, v, grad, div, edge_index, params):
    raise NotImplementedError("write your pallas kernel here")


# speedup vs baseline: 1.0748x; 1.0748x over previous
import functools

import jax
import jax.numpy as jnp
from jax.experimental import pallas as pl


def _mm_body(lhs_ref, rhs_ref, out_ref):
    out_ref[...] = jnp.dot(lhs_ref[...], rhs_ref[...],
                           preferred_element_type=jnp.float32)


def _big_matmul(mat, rhs, bm=200):
    n = mat.shape[0]
    k = rhs.shape[1]
    return pl.pallas_call(
        _mm_body,
        grid=(n // bm,),
        in_specs=[
            pl.BlockSpec((bm, n), lambda i: (i, 0)),
            pl.BlockSpec((n, k), lambda i: (0, 0)),
        ],
        out_specs=pl.BlockSpec((bm, k), lambda i: (i, 0)),
        out_shape=jax.ShapeDtypeStruct((n, k), jnp.float32),
    )(mat, rhs)


def _s1_body(x_ref, w_ref, b_ref, out_ref):
    out_ref[...] = jax.nn.relu(
        jnp.dot(x_ref[...], w_ref[...], preferred_element_type=jnp.float32)
        + b_ref[...])


def _s1(x, wt, b2, bm=2000):
    n, c = x.shape
    co = wt.shape[1]
    return pl.pallas_call(
        _s1_body,
        grid=(n // bm,),
        in_specs=[
            pl.BlockSpec((bm, c), lambda i: (i, 0)),
            pl.BlockSpec((c, co), lambda i: (0, 0)),
            pl.BlockSpec((1, co), lambda i: (0, 0)),
        ],
        out_specs=pl.BlockSpec((bm, co), lambda i: (i, 0)),
        out_shape=jax.ShapeDtypeStruct((n, co), jnp.float32),
    )(x, wt, b2)


def _xpath_body(x_ref, a_ref, v_ref, xmax_ref,
                ws_ref, bs_ref, wat_ref, bat_ref,
                wv_ref, bv_ref, wg_ref, bg_ref,
                lng_ref, lnb_ref, out_ref):
    x = x_ref[...]
    a = a_ref[...]
    v = v_ref[...]
    vn = v / (jnp.sqrt(jnp.sum(v * v, axis=1, keepdims=True)) + 1e-8)
    x_cat = jnp.concatenate([x, a, v - a, vn], axis=1)
    h = jax.nn.relu(
        jnp.dot(x_cat, ws_ref[...], preferred_element_type=jnp.float32)
        + bs_ref[...])
    h = xmax_ref[...] + h
    h = jnp.dot(h, wat_ref[...], preferred_element_type=jnp.float32) + bat_ref[...]
    vals = jnp.dot(h, wv_ref[...], preferred_element_type=jnp.float32) + bv_ref[...]
    gates = jax.nn.sigmoid(
        jnp.dot(h, wg_ref[...], preferred_element_type=jnp.float32) + bg_ref[...])
    gv = vals * gates
    c = out_ref.shape[1]
    x4 = 0.25 * (gv[:, :c] + gv[:, c:2 * c] + gv[:, 2 * c:3 * c] + gv[:, 3 * c:])
    xp = x4 + jnp.max(x4, axis=1, keepdims=True)
    mu = jnp.mean(xp, axis=1, keepdims=True)
    var = jnp.mean((xp - mu) ** 2, axis=1, keepdims=True)
    out_ref[...] = ((xp - mu) / jnp.sqrt(var + 1e-5)) * lng_ref[...] + lnb_ref[...]


def _xpath(x, a, v, x_max, ws, bs, wat, bat, wv, bv, wg, bg, lng, lnb, bm=2000):
    n, c = x.shape
    full = lambda arr: pl.BlockSpec(arr.shape, lambda i: (0,) * arr.ndim)
    row = lambda arr: pl.BlockSpec((bm, arr.shape[1]), lambda i: (i, 0))
    return pl.pallas_call(
        _xpath_body,
        grid=(n // bm,),
        in_specs=[row(x), row(a), row(v), row(x_max),
                  full(ws), full(bs), full(wat), full(bat),
                  full(wv), full(bv), full(wg), full(bg),
                  full(lng), full(lnb)],
        out_specs=pl.BlockSpec((bm, c), lambda i: (i, 0)),
        out_shape=jax.ShapeDtypeStruct((n, c), jnp.float32),
    )(x, a, v, x_max, ws, bs, wat, bat, wv, bv, wg, bg, lng, lnb)


def _vpath_body(v_ref, c_ref, d_ref, g_ref,
                wm_ref, bm_ref, wat_ref, bat_ref,
                wv_ref, bv_ref, wg_ref, bg_ref, out_ref):
    v = v_ref[...]
    hodge = c_ref[...] + d_ref[...]
    v_cat = jnp.concatenate([v, hodge, g_ref[...]], axis=1)
    h = jax.nn.relu(
        jnp.dot(v_cat, wm_ref[...], preferred_element_type=jnp.float32)
        + bm_ref[...])
    h = jnp.dot(h, wat_ref[...], preferred_element_type=jnp.float32) + bat_ref[...]
    vals = jnp.dot(h, wv_ref[...], preferred_element_type=jnp.float32) + bv_ref[...]
    gates = jax.nn.sigmoid(
        jnp.dot(h, wg_ref[...], preferred_element_type=jnp.float32) + bg_ref[...])
    gv = vals * gates
    c = out_ref.shape[1]
    v4 = 0.25 * (gv[:, :c] + gv[:, c:2 * c] + gv[:, 2 * c:3 * c] + gv[:, 3 * c:])
    out_ref[...] = v4 + jnp.mean(v4, axis=1, keepdims=True)


def _vpath(v, cc, d, g, wm, bmb, wat, bat, wv, bv, wg, bg, bm=2000):
    n, c = v.shape
    full = lambda arr: pl.BlockSpec(arr.shape, lambda i: (0,) * arr.ndim)
    row = lambda arr: pl.BlockSpec((bm, arr.shape[1]), lambda i: (i, 0))
    return pl.pallas_call(
        _vpath_body,
        grid=(n // bm,),
        in_specs=[row(v), row(cc), row(d), row(g),
                  full(wm), full(bmb), full(wat), full(bat),
                  full(wv), full(bv), full(wg), full(bg)],
        out_specs=pl.BlockSpec((bm, c), lambda i: (i, 0)),
        out_shape=jax.ShapeDtypeStruct((n, c), jnp.float32),
    )(v, cc, d, g, wm, bmb, wat, bat, wv, bv, wg, bg)


def _attn_combined(p):
    wv, bv = p["v"]["w"], p["v"]["b"]
    wo, bo = p["o"]["w"], p["o"]["b"]
    w = wv.T @ wo.T
    b = (bv @ wo.T + bo)[None, :]
    return w, b


def _glu_stacked(p):
    wv = p["wv"].transpose(2, 0, 1).reshape(p["wv"].shape[2], -1)
    bv = p["bv"].reshape(1, -1)
    wg = p["wg"].transpose(2, 0, 1).reshape(p["wg"].shape[2], -1)
    bg = p["bg"].reshape(1, -1)
    return wv, bv, wg, bg


def kernel(x, v, grad, div, edge_index, params):
    n = x.shape[0]

    w1t = params["s_mlp_max"][0]["w"].T
    b1 = params["s_mlp_max"][0]["b"][None, :]
    wst = params["s_mlp"][0]["w"].T
    bs = params["s_mlp"][0]["b"][None, :]
    wm_full = params["v_mlp"][0]["w"]
    half = wm_full.shape[1] // 2
    wmt = (wm_full[:, :half] - wm_full[:, half:]).T
    bmb = params["v_mlp"][0]["b"][None, :]
    wat_s, bat_s = _attn_combined(params["attn_s"])
    wat_v, bat_v = _attn_combined(params["attn_v"])
    wv_s, bv_s, wg_s, bg_s = _glu_stacked(params["glu_s"])
    wv_v, bv_v, wg_v, bg_v = _glu_stacked(params["glu_v"])
    lng = params["ln"]["g"][None, :]
    lnb = params["ln"]["b"][None, :]

    s1 = _s1(x, w1t, b1)
    msg = s1[edge_index[1]]
    x_max = jax.ops.segment_max(msg, edge_index[0], num_segments=n)
    x_max = jnp.where(jnp.isfinite(x_max), x_max, 0.0)

    b_ = _big_matmul(grad, v)
    ad = _big_matmul(div, jnp.concatenate([v, b_], axis=1))
    a, d = ad[:, :128], ad[:, 128:]

    x5 = _xpath(x, a, v, x_max, wst, bs, wat_s, bat_s,
                wv_s, bv_s, wg_s, bg_s, lng, lnb)

    cg = _big_matmul(grad, jnp.concatenate([a, x5], axis=1))
    c_, g_ = cg[:, :128], cg[:, 128:]

    v_out = _vpath(v, c_, d, g_, wmt, bmb, wat_v, bat_v,
                   wv_v, bv_v, wg_v, bg_v)
    return (x5, v_out)
